# baseline (speedup 1.0000x reference)
import functools

import jax
import jax.numpy as jnp
from jax import lax
from jax.experimental import pallas as pl
from jax.experimental.pallas import tpu as pltpu
from jax.experimental.pallas import tpu_sc as plsc

N = 10000
E = 320000
D = 128

NC = 2
NS = 16
NW = NC * NS
CHUNK = 128
CPT = 80
NBUF = 2
GRP = 16
NGRP = CPT // GRP
EPT = CPT * CHUNK
E_PAD = NW * EPT
N_ACC = 10240
RPT = N_ACC // NS
ACC_ROWS = N_ACC
RPT_E = ACC_ROWS // NS
DUMMY = N

_mesh = plsc.VectorSubcoreMesh(core_axis_name="c", subcore_axis_name="s")


@functools.partial(
    pl.kernel,
    out_type=jax.ShapeDtypeStruct((NC, N_ACC), jnp.float32),
    mesh=_mesh,
    scratch_types=[
        pltpu.VMEM((CPT, CHUNK), jnp.int32),
        pltpu.VMEM((CHUNK,), jnp.float32),
        pltpu.VMEM_SHARED((N_ACC,), jnp.float32),
    ],
)
def _deg_kernel(dst_hbm, ones_hbm, zeros_hbm, out_hbm, idx_v, ones_v, hist_sh):
    c = lax.axis_index("c")
    s = lax.axis_index("s")
    wid = c * NS + s

    pltpu.sync_copy(ones_hbm, ones_v)
    pltpu.sync_copy(zeros_hbm, hist_sh.at[pl.ds(s * RPT, RPT)])
    plsc.subcore_barrier()

    pltpu.sync_copy(dst_hbm.at[wid], idx_v)

    @pl.loop(0, CPT)
    def _(j):
        pltpu.sync_copy(ones_v, hist_sh.at[idx_v.at[j]], add=True)

    plsc.subcore_barrier()
    pltpu.sync_copy(
        hist_sh.at[pl.ds(s * RPT, RPT)], out_hbm.at[c, pl.ds(s * RPT, RPT)]
    )


@functools.partial(
    pl.kernel,
    out_type=jax.ShapeDtypeStruct((NC, N_ACC, D), jnp.float32),
    mesh=_mesh,
    scratch_types=[
        pltpu.VMEM((CPT, CHUNK), jnp.int32),
        pltpu.VMEM((CPT, CHUNK), jnp.int32),
        pltpu.VMEM((CHUNK, D), jnp.float32),
        pltpu.VMEM_SHARED((ACC_ROWS, D), jnp.float32),
    ],
)
def _edge_kernel(src_hbm, dst_hbm, g_hbm, zeros_hbm, out_hbm, si_v, di_v, buf_v, acc_sh):
    c = lax.axis_index("c")
    s = lax.axis_index("s")
    wid = c * NS + s

    pltpu.sync_copy(zeros_hbm, acc_sh.at[pl.ds(s * RPT_E, RPT_E)])

    plsc.subcore_barrier()

    pltpu.sync_copy(src_hbm.at[wid], si_v)
    pltpu.sync_copy(dst_hbm.at[wid], di_v)

    @pl.loop(0, CPT)
    def _(j):
        pltpu.sync_copy(g_hbm.at[si_v.at[j]], buf_v)
        pltpu.sync_copy(buf_v, acc_sh.at[di_v.at[j]], add=True)

    plsc.subcore_barrier()
    pltpu.sync_copy(
        acc_sh.at[pl.ds(s * RPT_E, RPT_E)], out_hbm.at[c, pl.ds(s * RPT_E, RPT_E)]
    )


BLK = 1024


def _mm_body(hist_ref, x_ref, w_ref, g_ref, dinv_ref):
    deg = hist_ref[0, :] + hist_ref[1, :] + 1.0
    dinv = lax.rsqrt(deg)
    h = jnp.dot(x_ref[...], w_ref[...], preferred_element_type=jnp.float32)
    g_ref[...] = h * dinv[:, None]
    dinv_ref[...] = dinv[None, :]


def _fin_body(acc_ref, g_ref, dinv_ref, b_ref, o_ref):
    t = (acc_ref[0] + acc_ref[1] + g_ref[...]) * dinv_ref[0, :][:, None]
    t = t + b_ref[0, :][None, :]
    o_ref[...] = t * jax.nn.sigmoid(t)


@jax.jit
def kernel(x, edge_index, W, b):
    src = edge_index[0]
    dst = edge_index[1]
    pad = E_PAD - E
    pad_iota = jnp.arange(pad, dtype=jnp.int32)
    src_p = jnp.concatenate(
        [src, jnp.full((pad,), N, jnp.int32)]
    ).reshape(NW, CPT, CHUNK)
    dst_p = jnp.concatenate(
        [dst, pad_iota % ACC_ROWS]
    ).reshape(NW, CPT, CHUNK)
    dst_deg = jnp.concatenate(
        [dst, DUMMY + pad_iota % (N_ACC - N)]
    ).reshape(NW, CPT, CHUNK)

    ones1 = jnp.ones((CHUNK,), jnp.float32)
    zeros1 = jnp.zeros((RPT,), jnp.float32)
    zeros2 = jnp.zeros((RPT_E, D), jnp.float32)

    hist = _deg_kernel(dst_deg, ones1, zeros1)

    x_p = jnp.concatenate([x, jnp.zeros((N_ACC - N, D), x.dtype)], axis=0)

    g, dinv = pl.pallas_call(
        _mm_body,
        grid=(N_ACC // BLK,),
        in_specs=[
            pl.BlockSpec((2, BLK), lambda i: (0, i)),
            pl.BlockSpec((BLK, D), lambda i: (i, 0)),
            pl.BlockSpec((D, D), lambda i: (0, 0)),
        ],
        out_specs=[
            pl.BlockSpec((BLK, D), lambda i: (i, 0)),
            pl.BlockSpec((1, BLK), lambda i: (0, i)),
        ],
        out_shape=[
            jax.ShapeDtypeStruct((N_ACC, D), jnp.float32),
            jax.ShapeDtypeStruct((1, N_ACC), jnp.float32),
        ],
    )(hist, x_p, W)

    acc = _edge_kernel(src_p, dst_p, g, zeros2)

    out = pl.pallas_call(
        _fin_body,
        grid=(N_ACC // BLK,),
        in_specs=[
            pl.BlockSpec((2, BLK, D), lambda i: (0, i, 0)),
            pl.BlockSpec((BLK, D), lambda i: (i, 0)),
            pl.BlockSpec((1, BLK), lambda i: (0, i)),
            pl.BlockSpec((1, D), lambda i: (0, 0)),
        ],
        out_specs=pl.BlockSpec((BLK, D), lambda i: (i, 0)),
        out_shape=jax.ShapeDtypeStruct((N_ACC, D), jnp.float32),
    )(acc, g, dinv, b.reshape(1, D))

    return out[:N]


# speedup vs baseline: 1.1324x; 1.1324x over previous
import functools

import jax
import jax.numpy as jnp
from jax import lax
from jax.experimental import pallas as pl
from jax.experimental.pallas import tpu as pltpu
from jax.experimental.pallas import tpu_sc as plsc

N = 10000
E = 320000
D = 128

NC = 2
NS = 16
NW = NC * NS
CHUNK = 128
CPT = 80
NBUF = 2
GRP = 16
NGRP = CPT // GRP
EPT = CPT * CHUNK
E_PAD = NW * EPT
N_ACC = 10240
RPT = N_ACC // NS
ACC_ROWS = N_ACC
RPT_E = ACC_ROWS // NS
DUMMY = N

_mesh = plsc.VectorSubcoreMesh(core_axis_name="c", subcore_axis_name="s")


@functools.partial(
    pl.kernel,
    out_type=jax.ShapeDtypeStruct((NC, N_ACC), jnp.float32),
    mesh=_mesh,
    scratch_types=[
        pltpu.VMEM((CPT, CHUNK), jnp.int32),
        pltpu.VMEM((CHUNK,), jnp.float32),
        pltpu.VMEM_SHARED((N_ACC,), jnp.float32),
    ],
)
def _deg_kernel(dst_hbm, ones_hbm, zeros_hbm, out_hbm, idx_v, ones_v, hist_sh):
    c = lax.axis_index("c")
    s = lax.axis_index("s")
    wid = c * NS + s

    pltpu.sync_copy(ones_hbm, ones_v)
    pltpu.sync_copy(zeros_hbm, hist_sh.at[pl.ds(s * RPT, RPT)])
    plsc.subcore_barrier()

    pltpu.sync_copy(dst_hbm.at[wid], idx_v)

    @pl.loop(0, CPT)
    def _(j):
        pltpu.sync_copy(ones_v, hist_sh.at[idx_v.at[j]], add=True)

    plsc.subcore_barrier()
    pltpu.sync_copy(
        hist_sh.at[pl.ds(s * RPT, RPT)], out_hbm.at[c, pl.ds(s * RPT, RPT)]
    )


@functools.partial(
    pl.kernel,
    out_type=jax.ShapeDtypeStruct((NC, N_ACC, D), jnp.float32),
    mesh=_mesh,
    scratch_types=[
        pltpu.VMEM((CPT, CHUNK), jnp.int32),
        pltpu.VMEM((CPT, CHUNK), jnp.int32),
        pltpu.VMEM((CHUNK, D), jnp.float32),
        pltpu.VMEM_SHARED((ACC_ROWS, D), jnp.float32),
    ],
)
def _edge_kernel(src_hbm, dst_hbm, g_hbm, zeros_hbm, out_hbm, si_v, di_v, buf_v, acc_sh):
    c = lax.axis_index("c")
    s = lax.axis_index("s")
    wid = c * NS + s

    pltpu.sync_copy(zeros_hbm, acc_sh.at[pl.ds(s * RPT_E, RPT_E)])

    plsc.subcore_barrier()

    pltpu.sync_copy(src_hbm.at[wid], si_v)
    pltpu.sync_copy(dst_hbm.at[wid], di_v)

    @pl.loop(0, CPT)
    def _(j):
        pltpu.sync_copy(g_hbm.at[si_v.at[j]], buf_v)
        pltpu.sync_copy(buf_v, acc_sh.at[di_v.at[j]], add=True)

    plsc.subcore_barrier()
    pltpu.sync_copy(
        acc_sh.at[pl.ds(s * RPT_E, RPT_E)], out_hbm.at[c, pl.ds(s * RPT_E, RPT_E)]
    )


BLK = 1024


def _mm_body(hist_ref, x_ref, w_ref, g_ref, dinv_ref):
    deg = hist_ref[0, :] + hist_ref[1, :] + 1.0
    dinv = lax.rsqrt(deg)
    h = jnp.dot(x_ref[...], w_ref[...], preferred_element_type=jnp.float32)
    g_ref[...] = h * dinv[:, None]
    dinv_ref[...] = dinv[None, :]


def _fin_body(acc_ref, g_ref, dinv_ref, b_ref, o_ref):
    t = (acc_ref[0] + acc_ref[1] + g_ref[...]) * dinv_ref[0, :][:, None]
    t = t + b_ref[0, :][None, :]
    o_ref[...] = t * jax.nn.sigmoid(t)


@jax.jit
def kernel(x, edge_index, W, b):
    ept_real = E // NW
    padt = EPT - ept_real
    tvec = jnp.arange(NW, dtype=jnp.int32)[:, None]
    pvec = jnp.arange(padt, dtype=jnp.int32)[None, :]
    src_t = edge_index[0].reshape(NW, ept_real)
    dst_t = edge_index[1].reshape(NW, ept_real)
    src_pad = jnp.broadcast_to(
        jnp.int32(N), (NW, padt)
    )
    src_p = jnp.concatenate([src_t, src_pad], axis=1).reshape(NW, CPT, CHUNK)
    dst_p = jnp.concatenate(
        [dst_t, (tvec * 320 + pvec) % ACC_ROWS], axis=1
    ).reshape(NW, CPT, CHUNK)
    dst_deg = jnp.concatenate(
        [dst_t, DUMMY + (tvec * 15 + pvec) % (N_ACC - N)], axis=1
    ).reshape(NW, CPT, CHUNK)

    ones1 = jnp.ones((CHUNK,), jnp.float32)
    zeros1 = jnp.zeros((RPT,), jnp.float32)
    zeros2 = jnp.zeros((RPT_E, D), jnp.float32)

    hist = _deg_kernel(dst_deg, ones1, zeros1)

    x_p = jnp.concatenate([x, jnp.zeros((N_ACC - N, D), x.dtype)], axis=0)

    g, dinv = pl.pallas_call(
        _mm_body,
        grid=(N_ACC // BLK,),
        in_specs=[
            pl.BlockSpec((2, BLK), lambda i: (0, i)),
            pl.BlockSpec((BLK, D), lambda i: (i, 0)),
            pl.BlockSpec((D, D), lambda i: (0, 0)),
        ],
        out_specs=[
            pl.BlockSpec((BLK, D), lambda i: (i, 0)),
            pl.BlockSpec((1, BLK), lambda i: (0, i)),
        ],
        out_shape=[
            jax.ShapeDtypeStruct((N_ACC, D), jnp.float32),
            jax.ShapeDtypeStruct((1, N_ACC), jnp.float32),
        ],
    )(hist, x_p, W)

    acc = _edge_kernel(src_p, dst_p, g, zeros2)

    out = pl.pallas_call(
        _fin_body,
        grid=(N_ACC // BLK,),
        in_specs=[
            pl.BlockSpec((2, BLK, D), lambda i: (0, i, 0)),
            pl.BlockSpec((BLK, D), lambda i: (i, 0)),
            pl.BlockSpec((1, BLK), lambda i: (0, i)),
            pl.BlockSpec((1, D), lambda i: (0, 0)),
        ],
        out_specs=pl.BlockSpec((BLK, D), lambda i: (i, 0)),
        out_shape=jax.ShapeDtypeStruct((N_ACC, D), jnp.float32),
    )(acc, g, dinv, b.reshape(1, D))

    return out[:N]


# speedup vs baseline: 2.3890x; 2.1096x over previous
import functools

import jax
import jax.numpy as jnp
from jax import lax
from jax.experimental import pallas as pl
from jax.experimental.pallas import tpu as pltpu
from jax.experimental.pallas import tpu_sc as plsc

N = 10000
E = 320000
D = 128

NC = 2
NS = 16
NW = NC * NS
CHUNK = 128
CPT = 80
NBUF = 2
GRP = 16
NGRP = CPT // GRP
EPT = CPT * CHUNK
E_PAD = NW * EPT
N_ACC = 10240
RPT = N_ACC // NS
ACC_ROWS = N_ACC
RPT_E = ACC_ROWS // NS
DUMMY = N

_mesh = plsc.VectorSubcoreMesh(core_axis_name="c", subcore_axis_name="s")


@functools.partial(
    pl.kernel,
    out_type=jax.ShapeDtypeStruct((NC, N_ACC), jnp.float32),
    mesh=_mesh,
    scratch_types=[
        pltpu.VMEM((CPT, CHUNK), jnp.int32),
        pltpu.VMEM((CHUNK,), jnp.float32),
        pltpu.VMEM_SHARED((N_ACC,), jnp.float32),
    ],
)
def _deg_kernel(dst_hbm, ones_hbm, zeros_hbm, out_hbm, idx_v, ones_v, hist_sh):
    c = lax.axis_index("c")
    s = lax.axis_index("s")
    wid = c * NS + s

    pltpu.sync_copy(ones_hbm, ones_v)
    pltpu.sync_copy(zeros_hbm, hist_sh.at[pl.ds(s * RPT, RPT)])
    plsc.subcore_barrier()

    pltpu.sync_copy(dst_hbm.at[wid], idx_v)

    @pl.loop(0, CPT)
    def _(j):
        pltpu.sync_copy(ones_v, hist_sh.at[idx_v.at[j]], add=True)

    plsc.subcore_barrier()
    pltpu.sync_copy(
        hist_sh.at[pl.ds(s * RPT, RPT)], out_hbm.at[c, pl.ds(s * RPT, RPT)]
    )


@functools.partial(
    pl.kernel,
    out_type=jax.ShapeDtypeStruct((NC, N_ACC, D), jnp.float32),
    mesh=_mesh,
    scratch_types=[
        pltpu.VMEM((CPT, CHUNK), jnp.int32),
        pltpu.VMEM((CPT, CHUNK), jnp.int32),
        pltpu.VMEM((CHUNK, D), jnp.float32),
        pltpu.VMEM_SHARED((ACC_ROWS, D), jnp.float32),
    ],
)
def _edge_kernel(src_hbm, dst_hbm, g_hbm, zeros_hbm, out_hbm, si_v, di_v, buf_v, acc_sh):
    c = lax.axis_index("c")
    s = lax.axis_index("s")
    wid = c * NS + s

    pltpu.sync_copy(zeros_hbm, acc_sh.at[pl.ds(s * RPT_E, RPT_E)])

    plsc.subcore_barrier()

    pltpu.sync_copy(src_hbm.at[wid], si_v)
    pltpu.sync_copy(dst_hbm.at[wid], di_v)

    @pl.loop(0, CPT)
    def _(j):
        pltpu.sync_copy(g_hbm.at[si_v.at[j]], buf_v)
        pltpu.sync_copy(buf_v, acc_sh.at[di_v.at[j]], add=True)

    plsc.subcore_barrier()
    pltpu.sync_copy(
        acc_sh.at[pl.ds(s * RPT_E, RPT_E)], out_hbm.at[c, pl.ds(s * RPT_E, RPT_E)]
    )


BLK = 1024


def _mm_body(hist_ref, x_ref, w_ref, g_ref, dinv_ref):
    deg = hist_ref[0, :] + hist_ref[1, :] + 1.0
    dinv = lax.rsqrt(deg)
    h = jnp.dot(x_ref[...], w_ref[...], preferred_element_type=jnp.float32)
    g_ref[...] = h * dinv[:, None]
    dinv_ref[...] = dinv[None, :]


def _fin_body(acc_ref, g_ref, dinv_ref, b_ref, o_ref):
    t = (acc_ref[0] + acc_ref[1] + g_ref[...]) * dinv_ref[0, :][:, None]
    t = t + b_ref[0, :][None, :]
    o_ref[...] = t * jax.nn.sigmoid(t)


@jax.jit
def kernel(x, edge_index, W, b):
    ept_real = E // NW
    padt = EPT - ept_real
    tvec = jnp.arange(NW, dtype=jnp.int32)[:, None]
    pvec = jnp.arange(padt, dtype=jnp.int32)[None, :]
    src_t = edge_index[0].reshape(NW, ept_real)
    dst_t = edge_index[1].reshape(NW, ept_real)
    src_pad = jnp.broadcast_to(N + pvec % (N_ACC - N), (NW, padt))
    src_p = jnp.concatenate([src_t, src_pad], axis=1).reshape(NW, CPT, CHUNK)
    dst_p = jnp.concatenate(
        [dst_t, (tvec * 320 + pvec) % ACC_ROWS], axis=1
    ).reshape(NW, CPT, CHUNK)
    dst_deg = jnp.concatenate(
        [dst_t, DUMMY + (tvec * 15 + pvec) % (N_ACC - N)], axis=1
    ).reshape(NW, CPT, CHUNK)

    ones1 = jnp.ones((CHUNK,), jnp.float32)
    zeros1 = jnp.zeros((RPT,), jnp.float32)
    zeros2 = jnp.zeros((RPT_E, D), jnp.float32)

    hist = _deg_kernel(dst_deg, ones1, zeros1)

    x_p = jnp.concatenate([x, jnp.zeros((N_ACC - N, D), x.dtype)], axis=0)

    g, dinv = pl.pallas_call(
        _mm_body,
        grid=(N_ACC // BLK,),
        in_specs=[
            pl.BlockSpec((2, BLK), lambda i: (0, i)),
            pl.BlockSpec((BLK, D), lambda i: (i, 0)),
            pl.BlockSpec((D, D), lambda i: (0, 0)),
        ],
        out_specs=[
            pl.BlockSpec((BLK, D), lambda i: (i, 0)),
            pl.BlockSpec((1, BLK), lambda i: (0, i)),
        ],
        out_shape=[
            jax.ShapeDtypeStruct((N_ACC, D), jnp.float32),
            jax.ShapeDtypeStruct((1, N_ACC), jnp.float32),
        ],
    )(hist, x_p, W)

    acc = _edge_kernel(src_p, dst_p, g, zeros2)

    out = pl.pallas_call(
        _fin_body,
        grid=(N_ACC // BLK,),
        in_specs=[
            pl.BlockSpec((2, BLK, D), lambda i: (0, i, 0)),
            pl.BlockSpec((BLK, D), lambda i: (i, 0)),
            pl.BlockSpec((1, BLK), lambda i: (0, i)),
            pl.BlockSpec((1, D), lambda i: (0, 0)),
        ],
        out_specs=pl.BlockSpec((BLK, D), lambda i: (i, 0)),
        out_shape=jax.ShapeDtypeStruct((N_ACC, D), jnp.float32),
    )(acc, g, dinv, b.reshape(1, D))

    return out[:N]


# speedup vs baseline: 2.6751x; 1.1198x over previous
import functools

import jax
import jax.numpy as jnp
from jax import lax
from jax.experimental import pallas as pl
from jax.experimental.pallas import tpu as pltpu
from jax.experimental.pallas import tpu_sc as plsc

N = 10000
E = 320000
D = 128

NC = 2
NS = 16
NW = NC * NS
CHUNK = 128
CPT = 80
NBUF = 2
GRP = 40
NGRP = CPT // GRP
EPT = CPT * CHUNK
E_PAD = NW * EPT
N_ACC = 10240
RPT = N_ACC // NS
ACC_ROWS = N_ACC
RPT_E = ACC_ROWS // NS
DUMMY = N

_mesh = plsc.VectorSubcoreMesh(core_axis_name="c", subcore_axis_name="s")


@functools.partial(
    pl.kernel,
    out_type=jax.ShapeDtypeStruct((NC, N_ACC), jnp.float32),
    mesh=_mesh,
    scratch_types=[
        pltpu.VMEM((CPT, CHUNK), jnp.int32),
        pltpu.VMEM((CHUNK,), jnp.float32),
        pltpu.VMEM_SHARED((N_ACC,), jnp.float32),
    ],
)
def _deg_kernel(dst_hbm, ones_hbm, zeros_hbm, out_hbm, idx_v, ones_v, hist_sh):
    c = lax.axis_index("c")
    s = lax.axis_index("s")
    wid = c * NS + s

    pltpu.sync_copy(ones_hbm, ones_v)
    pltpu.sync_copy(zeros_hbm, hist_sh.at[pl.ds(s * RPT, RPT)])
    plsc.subcore_barrier()

    pltpu.sync_copy(dst_hbm.at[wid], idx_v)

    @pl.loop(0, CPT)
    def _(j):
        pltpu.sync_copy(ones_v, hist_sh.at[idx_v.at[j]], add=True)

    plsc.subcore_barrier()
    pltpu.sync_copy(
        hist_sh.at[pl.ds(s * RPT, RPT)], out_hbm.at[c, pl.ds(s * RPT, RPT)]
    )


@functools.partial(
    pl.kernel,
    out_type=jax.ShapeDtypeStruct((NC, N_ACC, D), jnp.float32),
    mesh=_mesh,
    scratch_types=[
        pltpu.VMEM((GRP, CHUNK), jnp.int32),
        pltpu.VMEM((GRP, CHUNK), jnp.int32),
    ]
    + [pltpu.VMEM((CHUNK, D), jnp.float32) for _ in range(NBUF)]
    + [pltpu.SemaphoreType.DMA for _ in range(2 * NBUF)]
    + [pltpu.VMEM_SHARED((ACC_ROWS, D), jnp.float32)],
)
def _edge_kernel(src_hbm, dst_hbm, g_hbm, zeros_hbm, out_hbm, si_v, di_v, *rest):
    bufs = rest[:NBUF]
    gsems = rest[NBUF : 2 * NBUF]
    ssems = rest[2 * NBUF : 3 * NBUF]
    acc_sh = rest[3 * NBUF]
    c = lax.axis_index("c")
    s = lax.axis_index("s")
    wid = c * NS + s

    pltpu.sync_copy(zeros_hbm, acc_sh.at[pl.ds(s * RPT_E, RPT_E)])

    plsc.subcore_barrier()

    @pl.loop(0, NGRP)
    def _(gi):
        goff = pl.multiple_of(gi * GRP, GRP)
        pltpu.sync_copy(src_hbm.at[wid, pl.ds(goff, GRP)], si_v)
        pltpu.sync_copy(dst_hbm.at[wid, pl.ds(goff, GRP)], di_v)

        @pl.loop(0, GRP, step=NBUF)
        def _(j):
            gh = [
                pltpu.async_copy(g_hbm.at[si_v.at[j + b]], bufs[b], gsems[b])
                for b in range(NBUF)
            ]
            sh = []
            for b in range(NBUF):
                gh[b].wait()
                sh.append(
                    pltpu.async_copy(
                        bufs[b], acc_sh.at[di_v.at[j + b]], ssems[b], add=True
                    )
                )
            for b in range(NBUF):
                sh[b].wait()

    plsc.subcore_barrier()
    pltpu.sync_copy(
        acc_sh.at[pl.ds(s * RPT_E, RPT_E)], out_hbm.at[c, pl.ds(s * RPT_E, RPT_E)]
    )


BLK = 1024


def _mm_body(hist_ref, x_ref, w_ref, g_ref, dinv_ref):
    deg = hist_ref[0, :] + hist_ref[1, :] + 1.0
    dinv = lax.rsqrt(deg)
    h = jnp.dot(x_ref[...], w_ref[...], preferred_element_type=jnp.float32)
    g_ref[...] = h * dinv[:, None]
    dinv_ref[...] = dinv[None, :]


def _fin_body(acc_ref, g_ref, dinv_ref, b_ref, o_ref):
    t = (acc_ref[0] + acc_ref[1] + g_ref[...]) * dinv_ref[0, :][:, None]
    t = t + b_ref[0, :][None, :]
    o_ref[...] = t * jax.nn.sigmoid(t)


@jax.jit
def kernel(x, edge_index, W, b):
    ept_real = E // NW
    padt = EPT - ept_real
    tvec = jnp.arange(NW, dtype=jnp.int32)[:, None]
    pvec = jnp.arange(padt, dtype=jnp.int32)[None, :]
    src_t = edge_index[0].reshape(NW, ept_real)
    dst_t = edge_index[1].reshape(NW, ept_real)
    src_pad = jnp.broadcast_to(N + pvec % (N_ACC - N), (NW, padt))
    src_p = jnp.concatenate([src_t, src_pad], axis=1).reshape(NW, CPT, CHUNK)
    dst_p = jnp.concatenate(
        [dst_t, (tvec * 320 + pvec) % ACC_ROWS], axis=1
    ).reshape(NW, CPT, CHUNK)
    dst_deg = jnp.concatenate(
        [dst_t, DUMMY + (tvec * 15 + pvec) % (N_ACC - N)], axis=1
    ).reshape(NW, CPT, CHUNK)

    ones1 = jnp.ones((CHUNK,), jnp.float32)
    zeros1 = jnp.zeros((RPT,), jnp.float32)
    zeros2 = jnp.zeros((RPT_E, D), jnp.float32)

    hist = _deg_kernel(dst_deg, ones1, zeros1)

    x_p = jnp.concatenate([x, jnp.zeros((N_ACC - N, D), x.dtype)], axis=0)

    g, dinv = pl.pallas_call(
        _mm_body,
        grid=(N_ACC // BLK,),
        in_specs=[
            pl.BlockSpec((2, BLK), lambda i: (0, i)),
            pl.BlockSpec((BLK, D), lambda i: (i, 0)),
            pl.BlockSpec((D, D), lambda i: (0, 0)),
        ],
        out_specs=[
            pl.BlockSpec((BLK, D), lambda i: (i, 0)),
            pl.BlockSpec((1, BLK), lambda i: (0, i)),
        ],
        out_shape=[
            jax.ShapeDtypeStruct((N_ACC, D), jnp.float32),
            jax.ShapeDtypeStruct((1, N_ACC), jnp.float32),
        ],
    )(hist, x_p, W)

    acc = _edge_kernel(src_p, dst_p, g, zeros2)

    out = pl.pallas_call(
        _fin_body,
        grid=(N_ACC // BLK,),
        in_specs=[
            pl.BlockSpec((2, BLK, D), lambda i: (0, i, 0)),
            pl.BlockSpec((BLK, D), lambda i: (i, 0)),
            pl.BlockSpec((1, BLK), lambda i: (0, i)),
            pl.BlockSpec((1, D), lambda i: (0, 0)),
        ],
        out_specs=pl.BlockSpec((BLK, D), lambda i: (i, 0)),
        out_shape=jax.ShapeDtypeStruct((N_ACC, D), jnp.float32),
    )(acc, g, dinv, b.reshape(1, D))

    return out[:N]


# speedup vs baseline: 2.7375x; 1.0233x over previous
import functools

import jax
import jax.numpy as jnp
from jax import lax
from jax.experimental import pallas as pl
from jax.experimental.pallas import tpu as pltpu
from jax.experimental.pallas import tpu_sc as plsc

N = 10000
E = 320000
D = 128

NC = 2
NS = 16
NW = NC * NS
CHUNK = 128
CPT = 80
NBUF = 2
GRP = 40
NGRP = CPT // GRP
EPT = CPT * CHUNK
E_PAD = NW * EPT
N_ACC = 10240
RPT = N_ACC // NS
ACC_ROWS = N_ACC
RPT_E = ACC_ROWS // NS
DUMMY = N

_mesh = plsc.VectorSubcoreMesh(core_axis_name="c", subcore_axis_name="s")


@functools.partial(
    pl.kernel,
    out_type=jax.ShapeDtypeStruct((NC, N_ACC), jnp.float32),
    mesh=_mesh,
    scratch_types=[
        pltpu.VMEM((CPT, CHUNK), jnp.int32),
        pltpu.VMEM((CHUNK,), jnp.float32),
        pltpu.SemaphoreType.DMA,
        pltpu.VMEM_SHARED((N_ACC,), jnp.float32),
    ],
)
def _deg_kernel(dst_hbm, ones_hbm, zeros_hbm, out_hbm, idx_v, ones_v, sem, hist_sh):
    c = lax.axis_index("c")
    s = lax.axis_index("s")
    wid = c * NS + s

    pltpu.sync_copy(ones_hbm, ones_v)
    pltpu.sync_copy(zeros_hbm, hist_sh.at[pl.ds(s * RPT, RPT)])
    plsc.subcore_barrier()

    pltpu.sync_copy(dst_hbm.at[wid], idx_v)

    @pl.loop(0, CPT)
    def _(j):
        pltpu.async_copy(ones_v, hist_sh.at[idx_v.at[j]], sem, add=True)

    @pl.loop(0, CPT)
    def _(j):
        pltpu.make_async_copy(ones_v, hist_sh.at[idx_v.at[j]], sem).wait()

    plsc.subcore_barrier()
    pltpu.sync_copy(
        hist_sh.at[pl.ds(s * RPT, RPT)], out_hbm.at[c, pl.ds(s * RPT, RPT)]
    )


@functools.partial(
    pl.kernel,
    out_type=jax.ShapeDtypeStruct((NC, N_ACC, D), jnp.float32),
    mesh=_mesh,
    scratch_types=[
        pltpu.VMEM((GRP, CHUNK), jnp.int32),
        pltpu.VMEM((GRP, CHUNK), jnp.int32),
    ]
    + [pltpu.VMEM((CHUNK, D), jnp.float32) for _ in range(NBUF)]
    + [pltpu.SemaphoreType.DMA for _ in range(2 * NBUF)]
    + [pltpu.VMEM_SHARED((ACC_ROWS, D), jnp.float32)],
)
def _edge_kernel(src_hbm, dst_hbm, g_hbm, zeros_hbm, out_hbm, si_v, di_v, *rest):
    bufs = rest[:NBUF]
    gsems = rest[NBUF : 2 * NBUF]
    ssems = rest[2 * NBUF : 3 * NBUF]
    acc_sh = rest[3 * NBUF]
    c = lax.axis_index("c")
    s = lax.axis_index("s")
    wid = c * NS + s

    pltpu.sync_copy(zeros_hbm, acc_sh.at[pl.ds(s * RPT_E, RPT_E)])

    plsc.subcore_barrier()

    @pl.loop(0, NGRP)
    def _(gi):
        goff = pl.multiple_of(gi * GRP, GRP)
        pltpu.sync_copy(src_hbm.at[wid, pl.ds(goff, GRP)], si_v)
        pltpu.sync_copy(dst_hbm.at[wid, pl.ds(goff, GRP)], di_v)

        @pl.loop(0, GRP, step=NBUF)
        def _(j):
            gh = [
                pltpu.async_copy(g_hbm.at[si_v.at[j + b]], bufs[b], gsems[b])
                for b in range(NBUF)
            ]
            sh = []
            for b in range(NBUF):
                gh[b].wait()
                sh.append(
                    pltpu.async_copy(
                        bufs[b], acc_sh.at[di_v.at[j + b]], ssems[b], add=True
                    )
                )
            for b in range(NBUF):
                sh[b].wait()

    plsc.subcore_barrier()
    pltpu.sync_copy(
        acc_sh.at[pl.ds(s * RPT_E, RPT_E)], out_hbm.at[c, pl.ds(s * RPT_E, RPT_E)]
    )


BLK = 1024


def _mm_body(hist_ref, x_ref, w_ref, g_ref, dinv_ref):
    deg = hist_ref[0, :] + hist_ref[1, :] + 1.0
    dinv = lax.rsqrt(deg)
    h = jnp.dot(x_ref[...], w_ref[...], preferred_element_type=jnp.float32)
    g_ref[...] = h * dinv[:, None]
    dinv_ref[...] = dinv[None, :]


def _fin_body(acc_ref, g_ref, dinv_ref, b_ref, o_ref):
    t = (acc_ref[0] + acc_ref[1] + g_ref[...]) * dinv_ref[0, :][:, None]
    t = t + b_ref[0, :][None, :]
    o_ref[...] = t * jax.nn.sigmoid(t)


@jax.jit
def kernel(x, edge_index, W, b):
    ept_real = E // NW
    padt = EPT - ept_real
    tvec = jnp.arange(NW, dtype=jnp.int32)[:, None]
    pvec = jnp.arange(padt, dtype=jnp.int32)[None, :]
    src_t = edge_index[0].reshape(NW, ept_real)
    dst_t = edge_index[1].reshape(NW, ept_real)
    src_pad = jnp.broadcast_to(N + pvec % (N_ACC - N), (NW, padt))
    src_p = jnp.concatenate([src_t, src_pad], axis=1).reshape(NW, CPT, CHUNK)
    dst_p = jnp.concatenate(
        [dst_t, (tvec * 320 + pvec) % ACC_ROWS], axis=1
    ).reshape(NW, CPT, CHUNK)
    dst_deg = jnp.concatenate(
        [dst_t, DUMMY + (tvec * 15 + pvec) % (N_ACC - N)], axis=1
    ).reshape(NW, CPT, CHUNK)

    ones1 = jnp.ones((CHUNK,), jnp.float32)
    zeros1 = jnp.zeros((RPT,), jnp.float32)
    zeros2 = jnp.zeros((RPT_E, D), jnp.float32)

    hist = _deg_kernel(dst_deg, ones1, zeros1)

    x_p = jnp.concatenate([x, jnp.zeros((N_ACC - N, D), x.dtype)], axis=0)

    g, dinv = pl.pallas_call(
        _mm_body,
        grid=(N_ACC // BLK,),
        in_specs=[
            pl.BlockSpec((2, BLK), lambda i: (0, i)),
            pl.BlockSpec((BLK, D), lambda i: (i, 0)),
            pl.BlockSpec((D, D), lambda i: (0, 0)),
        ],
        out_specs=[
            pl.BlockSpec((BLK, D), lambda i: (i, 0)),
            pl.BlockSpec((1, BLK), lambda i: (0, i)),
        ],
        out_shape=[
            jax.ShapeDtypeStruct((N_ACC, D), jnp.float32),
            jax.ShapeDtypeStruct((1, N_ACC), jnp.float32),
        ],
    )(hist, x_p, W)

    acc = _edge_kernel(src_p, dst_p, g, zeros2)

    out = pl.pallas_call(
        _fin_body,
        grid=(N_ACC // BLK,),
        in_specs=[
            pl.BlockSpec((2, BLK, D), lambda i: (0, i, 0)),
            pl.BlockSpec((BLK, D), lambda i: (i, 0)),
            pl.BlockSpec((1, BLK), lambda i: (0, i)),
            pl.BlockSpec((1, D), lambda i: (0, 0)),
        ],
        out_specs=pl.BlockSpec((BLK, D), lambda i: (i, 0)),
        out_shape=jax.ShapeDtypeStruct((N_ACC, D), jnp.float32),
    )(acc, g, dinv, b.reshape(1, D))

    return out[:N]


# speedup vs baseline: 2.7686x; 1.0113x over previous
import functools

import jax
import jax.numpy as jnp
from jax import lax
from jax.experimental import pallas as pl
from jax.experimental.pallas import tpu as pltpu
from jax.experimental.pallas import tpu_sc as plsc

N = 10000
E = 320000
D = 128

NC = 2
NS = 16
NW = NC * NS
CHUNK = 128
CPT = 80
NBUF = 2
GRP = 40
NGRP = CPT // GRP
EPT = CPT * CHUNK
E_PAD = NW * EPT
N_ACC = 10240
RPT = N_ACC // NS
ACC_ROWS = N_ACC
RPT_E = ACC_ROWS // NS
DUMMY = N

_mesh = plsc.VectorSubcoreMesh(core_axis_name="c", subcore_axis_name="s")


@functools.partial(
    pl.kernel,
    out_type=jax.ShapeDtypeStruct((NC, N_ACC), jnp.float32),
    mesh=_mesh,
    scratch_types=[
        pltpu.VMEM((CPT, CHUNK), jnp.int32),
        pltpu.VMEM((CHUNK,), jnp.float32),
        pltpu.SemaphoreType.DMA,
        pltpu.VMEM_SHARED((N_ACC,), jnp.float32),
    ],
)
def _deg_kernel(dst_hbm, ones_hbm, zeros_hbm, out_hbm, idx_v, ones_v, sem, hist_sh):
    c = lax.axis_index("c")
    s = lax.axis_index("s")
    wid = c * NS + s

    pltpu.sync_copy(ones_hbm, ones_v)
    pltpu.sync_copy(zeros_hbm, hist_sh.at[pl.ds(s * RPT, RPT)])
    plsc.subcore_barrier()

    pltpu.sync_copy(dst_hbm.at[wid], idx_v)

    @pl.loop(0, CPT)
    def _(j):
        pltpu.async_copy(ones_v, hist_sh.at[idx_v.at[j]], sem, add=True)

    @pl.loop(0, CPT)
    def _(j):
        pltpu.make_async_copy(ones_v, hist_sh.at[idx_v.at[j]], sem).wait()

    plsc.subcore_barrier()
    pltpu.sync_copy(
        hist_sh.at[pl.ds(s * RPT, RPT)], out_hbm.at[c, pl.ds(s * RPT, RPT)]
    )


@functools.partial(
    pl.kernel,
    out_type=jax.ShapeDtypeStruct((NC, N_ACC, D), jnp.float32),
    mesh=_mesh,
    scratch_types=[
        pltpu.VMEM((GRP, CHUNK), jnp.int32),
        pltpu.VMEM((GRP, CHUNK), jnp.int32),
    ]
    + [pltpu.VMEM((CHUNK, D), jnp.float32) for _ in range(NBUF)]
    + [pltpu.SemaphoreType.DMA for _ in range(2 * NBUF)]
    + [pltpu.VMEM_SHARED((ACC_ROWS, D), jnp.float32)],
)
def _edge_kernel(src_hbm, dst_hbm, g_hbm, zeros_hbm, out_hbm, si_v, di_v, *rest):
    bufs = rest[:NBUF]
    gsems = rest[NBUF : 2 * NBUF]
    ssems = rest[2 * NBUF : 3 * NBUF]
    acc_sh = rest[3 * NBUF]
    c = lax.axis_index("c")
    s = lax.axis_index("s")
    wid = c * NS + s

    pltpu.sync_copy(zeros_hbm, acc_sh.at[pl.ds(s * RPT_E, RPT_E)])

    plsc.subcore_barrier()

    @pl.loop(0, NGRP)
    def _(gi):
        goff = pl.multiple_of(gi * GRP, GRP)
        pltpu.sync_copy(src_hbm.at[wid, pl.ds(goff, GRP)], si_v)
        pltpu.sync_copy(dst_hbm.at[wid, pl.ds(goff, GRP)], di_v)

        for b in range(NBUF):
            pltpu.async_copy(g_hbm.at[si_v.at[b]], bufs[b], gsems[b])

        @pl.loop(0, GRP, step=NBUF)
        def _(j):
            for b in range(NBUF):
                pltpu.make_async_copy(
                    g_hbm.at[si_v.at[j + b]], bufs[b], gsems[b]
                ).wait()
                pltpu.async_copy(
                    bufs[b], acc_sh.at[di_v.at[j + b]], ssems[b], add=True
                )

            @pl.when(j + NBUF < GRP)
            def _():
                for b in range(NBUF):
                    pltpu.make_async_copy(
                        bufs[b], acc_sh.at[di_v.at[j + b]], ssems[b]
                    ).wait()
                    pltpu.async_copy(
                        g_hbm.at[si_v.at[j + NBUF + b]], bufs[b], gsems[b]
                    )

        for b in range(NBUF):
            pltpu.make_async_copy(
                bufs[b], acc_sh.at[di_v.at[GRP - NBUF + b]], ssems[b]
            ).wait()

    plsc.subcore_barrier()
    pltpu.sync_copy(
        acc_sh.at[pl.ds(s * RPT_E, RPT_E)], out_hbm.at[c, pl.ds(s * RPT_E, RPT_E)]
    )


BLK = 1024


def _mm_body(hist_ref, x_ref, w_ref, g_ref, dinv_ref):
    deg = hist_ref[0, :] + hist_ref[1, :] + 1.0
    dinv = lax.rsqrt(deg)
    h = jnp.dot(x_ref[...], w_ref[...], preferred_element_type=jnp.float32)
    g_ref[...] = h * dinv[:, None]
    dinv_ref[...] = dinv[None, :]


def _fin_body(acc_ref, g_ref, dinv_ref, b_ref, o_ref):
    t = (acc_ref[0] + acc_ref[1] + g_ref[...]) * dinv_ref[0, :][:, None]
    t = t + b_ref[0, :][None, :]
    o_ref[...] = t * jax.nn.sigmoid(t)


@jax.jit
def kernel(x, edge_index, W, b):
    ept_real = E // NW
    padt = EPT - ept_real
    tvec = jnp.arange(NW, dtype=jnp.int32)[:, None]
    pvec = jnp.arange(padt, dtype=jnp.int32)[None, :]
    src_t = edge_index[0].reshape(NW, ept_real)
    dst_t = edge_index[1].reshape(NW, ept_real)
    src_pad = jnp.broadcast_to(N + pvec % (N_ACC - N), (NW, padt))
    src_p = jnp.concatenate([src_t, src_pad], axis=1).reshape(NW, CPT, CHUNK)
    dst_p = jnp.concatenate(
        [dst_t, (tvec * 320 + pvec) % ACC_ROWS], axis=1
    ).reshape(NW, CPT, CHUNK)
    dst_deg = jnp.concatenate(
        [dst_t, DUMMY + (tvec * 15 + pvec) % (N_ACC - N)], axis=1
    ).reshape(NW, CPT, CHUNK)

    ones1 = jnp.ones((CHUNK,), jnp.float32)
    zeros1 = jnp.zeros((RPT,), jnp.float32)
    zeros2 = jnp.zeros((RPT_E, D), jnp.float32)

    hist = _deg_kernel(dst_deg, ones1, zeros1)

    x_p = jnp.concatenate([x, jnp.zeros((N_ACC - N, D), x.dtype)], axis=0)

    g, dinv = pl.pallas_call(
        _mm_body,
        grid=(N_ACC // BLK,),
        in_specs=[
            pl.BlockSpec((2, BLK), lambda i: (0, i)),
            pl.BlockSpec((BLK, D), lambda i: (i, 0)),
            pl.BlockSpec((D, D), lambda i: (0, 0)),
        ],
        out_specs=[
            pl.BlockSpec((BLK, D), lambda i: (i, 0)),
            pl.BlockSpec((1, BLK), lambda i: (0, i)),
        ],
        out_shape=[
            jax.ShapeDtypeStruct((N_ACC, D), jnp.float32),
            jax.ShapeDtypeStruct((1, N_ACC), jnp.float32),
        ],
    )(hist, x_p, W)

    acc = _edge_kernel(src_p, dst_p, g, zeros2)

    out = pl.pallas_call(
        _fin_body,
        grid=(N_ACC // BLK,),
        in_specs=[
            pl.BlockSpec((2, BLK, D), lambda i: (0, i, 0)),
            pl.BlockSpec((BLK, D), lambda i: (i, 0)),
            pl.BlockSpec((1, BLK), lambda i: (0, i)),
            pl.BlockSpec((1, D), lambda i: (0, 0)),
        ],
        out_specs=pl.BlockSpec((BLK, D), lambda i: (i, 0)),
        out_shape=jax.ShapeDtypeStruct((N_ACC, D), jnp.float32),
    )(acc, g, dinv, b.reshape(1, D))

    return out[:N]
